# Initial kernel scaffold; baseline (speedup 1.0000x reference)
import jax
import jax.numpy as jnp
from jax.experimental import pallas as pl


def kernel(x, edge_index, batch, emb, W1, b1, W2, b2, Wl, bl):
    raise NotImplementedError("write your pallas kernel here")


# speedup vs baseline: 22.4169x; 22.4169x over previous
import functools

import jax
import jax.numpy as jnp
from jax import lax
from jax.experimental import pallas as pl
from jax.experimental.pallas import tpu as pltpu
from jax.experimental.pallas import tpu_sc as plsc

N = 50000
E = 800000
G = 512
VOCAB = 100000
D = 64
NCL = 2

NTILE = 16
NCORE = 2
CH = 128
EC_T = 392
EP = NTILE * EC_T * CH
NP = 53248
NC_T = NP // CH // NTILE
NC_W = NP // CH // (NTILE * NCORE)
NPT = NP // NTILE
GP = 528
DUM = N
BLK = 1024
HALF = D // 2


def _sc_prep_body(emb_h, x3_h, dstA_h, bA_h, z1_h,
                  h0_h, deg2_h, cnt2_h,
                  xv, rows, dstv, bv, ones_v, zg_v, deg_sh, cnt_sh, gsem):
    c = lax.axis_index("c")
    s = lax.axis_index("s")
    wid = c * NTILE + s

    for i in range(CH // 16):
        ones_v[pl.ds(i * 16, 16)] = jnp.full((16,), 1.0, jnp.float32)

    pltpu.sync_copy(z1_h.at[pl.ds(s * NPT, NPT)], deg_sh.at[pl.ds(s * NPT, NPT)])

    @pl.when(s == 0)
    def _():
        for i in range(GP // 16):
            zg_v[pl.ds(i * 16, 16)] = jnp.zeros((16,), jnp.float32)
        pltpu.sync_copy(zg_v, cnt_sh)

    pltpu.sync_copy(x3_h.at[wid], xv)

    def emb_body(k, _):
        pltpu.async_copy(emb_h.at[xv.at[k]], rows, gsem).wait()
        pltpu.sync_copy(rows, h0_h.at[pl.ds(wid * NC_W * CH + k * CH, CH)])
        return 0

    lax.fori_loop(0, NC_W, emb_body, 0)

    plsc.subcore_barrier()

    pltpu.sync_copy(dstA_h.at[wid], dstv)

    def deg_body(k, _):
        pltpu.sync_copy(ones_v, deg_sh.at[dstv.at[k]], add=True)
        return 0

    lax.fori_loop(0, EC_T // 2, deg_body, 0)

    pltpu.sync_copy(bA_h.at[wid], bv)

    def cnt_body(k, _):
        pltpu.sync_copy(ones_v, cnt_sh.at[bv.at[k]], add=True)
        return 0

    lax.fori_loop(0, NC_W, cnt_body, 0)

    plsc.subcore_barrier()

    pltpu.sync_copy(deg_sh.at[pl.ds(s * NPT, NPT)],
                    deg2_h.at[c, pl.ds(s * NPT, NPT)])

    @pl.when(s == 0)
    def _():
        pltpu.sync_copy(cnt_sh, cnt2_h.at[c])


def _sc_prep(emb, x3, dstA, bA, z1):
    mesh = plsc.VectorSubcoreMesh(core_axis_name="c", subcore_axis_name="s")
    return pl.kernel(
        _sc_prep_body,
        out_type=[
            jax.ShapeDtypeStruct((NP, D), jnp.float32),
            jax.ShapeDtypeStruct((NCORE, NP), jnp.float32),
            jax.ShapeDtypeStruct((NCORE, GP), jnp.float32),
        ],
        mesh=mesh,
        scratch_types=[
            pltpu.VMEM((NC_W, CH), jnp.int32),
            pltpu.VMEM((CH, D), jnp.float32),
            pltpu.VMEM((EC_T // 2, CH), jnp.int32),
            pltpu.VMEM((NC_W, CH), jnp.int32),
            pltpu.VMEM((CH,), jnp.float32),
            pltpu.VMEM((GP,), jnp.float32),
            pltpu.VMEM_SHARED((NP,), jnp.float32),
            pltpu.VMEM_SHARED((GP,), jnp.float32),
            pltpu.SemaphoreType.DMA,
        ],
        name="sc_prep",
        compiler_params=pltpu.CompilerParams(use_tc_tiling_on_sc=False),
    )(emb, x3, dstA, bA, z1)


IB = 14
NG = EC_T // IB


def _sc_prop_body(glo_h, ghi_h, src3_h, dst3_h, z2_h,
                  alo_h, ahi_h,
                  srcv, dstv, r0, r1, acc_sh, gs0, gs1, isem):
    c = lax.axis_index("c")
    s = lax.axis_index("s")

    pltpu.sync_copy(z2_h.at[pl.ds(s * NPT, NPT)], acc_sh.at[pl.ds(s * NPT, NPT)])
    plsc.subcore_barrier()

    def run(g_h, a_h):
        pltpu.async_copy(src3_h.at[s, pl.ds(0, IB)], srcv.at[0], isem)
        pltpu.async_copy(dst3_h.at[s, pl.ds(0, IB)], dstv.at[0], isem)

        def group(g, _):
            buf = lax.rem(g, 2)
            pltpu.make_async_copy(src3_h.at[s, pl.ds(g * IB, IB)],
                                  srcv.at[buf], isem).wait()
            pltpu.make_async_copy(dst3_h.at[s, pl.ds(g * IB, IB)],
                                  dstv.at[buf], isem).wait()

            @pl.when(g + 1 < NG)
            def _():
                nb = lax.rem(g + 1, 2)
                pltpu.async_copy(src3_h.at[s, pl.ds((g + 1) * IB, IB)],
                                 srcv.at[nb], isem)
                pltpu.async_copy(dst3_h.at[s, pl.ds((g + 1) * IB, IB)],
                                 dstv.at[nb], isem)

            pltpu.async_copy(g_h.at[srcv.at[buf, 0]], r0, gs0)
            pltpu.async_copy(g_h.at[srcv.at[buf, 1]], r1, gs1)

            def body(j, _):
                k0 = 2 * j
                pltpu.make_async_copy(g_h.at[srcv.at[buf, k0]], r0, gs0).wait()
                pltpu.sync_copy(r0, acc_sh.at[dstv.at[buf, k0]], add=True)

                @pl.when(k0 + 2 < IB)
                def _():
                    pltpu.async_copy(g_h.at[srcv.at[buf, k0 + 2]], r0, gs0)

                pltpu.make_async_copy(g_h.at[srcv.at[buf, k0 + 1]], r1, gs1).wait()
                pltpu.sync_copy(r1, acc_sh.at[dstv.at[buf, k0 + 1]], add=True)

                @pl.when(k0 + 3 < IB)
                def _():
                    pltpu.async_copy(g_h.at[srcv.at[buf, k0 + 3]], r1, gs1)

                return 0

            lax.fori_loop(0, IB // 2, body, 0)
            return 0

        lax.fori_loop(0, NG, group, 0)
        plsc.subcore_barrier()
        pltpu.sync_copy(acc_sh.at[pl.ds(s * NPT, NPT)],
                        a_h.at[pl.ds(s * NPT, NPT)])

    @pl.when(c == 0)
    def _():
        run(glo_h, alo_h)

    @pl.when(c == 1)
    def _():
        run(ghi_h, ahi_h)


def _sc_prop(g_lo, g_hi, src3, dst3, z2):
    mesh = plsc.VectorSubcoreMesh(core_axis_name="c", subcore_axis_name="s")
    return pl.kernel(
        _sc_prop_body,
        out_type=[
            jax.ShapeDtypeStruct((NP, HALF), jnp.float32),
            jax.ShapeDtypeStruct((NP, HALF), jnp.float32),
        ],
        mesh=mesh,
        scratch_types=[
            pltpu.VMEM((2, IB, CH), jnp.int32),
            pltpu.VMEM((2, IB, CH), jnp.int32),
            pltpu.VMEM((CH, HALF), jnp.float32),
            pltpu.VMEM((CH, HALF), jnp.float32),
            pltpu.VMEM_SHARED((NP, HALF), jnp.float32),
            pltpu.SemaphoreType.DMA,
            pltpu.SemaphoreType.DMA,
            pltpu.SemaphoreType.DMA,
        ],
        name="sc_prop",
        compiler_params=pltpu.CompilerParams(use_tc_tiling_on_sc=False),
    )(g_lo, g_hi, src3, dst3, z2)


def _sc_pool_body(q2_h, b3_h, cnt2_h, blb_h, res_h,
                  bv, qv, sums_v, ca, cb, blv, res_v, sums_sh):
    c = lax.axis_index("c")
    s = lax.axis_index("s")

    @pl.when(s == 0)
    def _():
        for i in range(GP // 16):
            res_v[pl.ds(i * 16, 16)] = jnp.zeros((16,), jnp.float32)
        pltpu.sync_copy(res_v, sums_sh)

    pltpu.sync_copy(q2_h.at[c, pl.ds(s * NPT, NPT)], qv)
    pltpu.sync_copy(b3_h.at[s], bv)
    plsc.subcore_barrier()

    def body(k, _):
        pltpu.sync_copy(qv.at[pl.ds(k * CH, CH)], sums_sh.at[bv.at[k]], add=True)
        return 0

    lax.fori_loop(0, NC_T, body, 0)
    plsc.subcore_barrier()

    @pl.when(s == 0)
    def _():
        pltpu.sync_copy(sums_sh, sums_v)
        pltpu.sync_copy(cnt2_h.at[0], ca)
        pltpu.sync_copy(cnt2_h.at[1], cb)
        pltpu.sync_copy(blb_h.at[c], blv)
        bvec = blv[...]
        for i in range(GP // 16):
            d = pl.ds(i * 16, 16)
            cv = ca[d] + cb[d]
            res_v[d] = sums_v[d] / jnp.maximum(cv, 1.0) + bvec
        pltpu.sync_copy(res_v, res_h.at[c])


def _sc_pool(q2, b3, cnt2, blb):
    mesh = plsc.VectorSubcoreMesh(core_axis_name="c", subcore_axis_name="s")
    return pl.kernel(
        _sc_pool_body,
        out_type=jax.ShapeDtypeStruct((NCORE, GP), jnp.float32),
        mesh=mesh,
        scratch_types=[
            pltpu.VMEM((NC_T, CH), jnp.int32),
            pltpu.VMEM((NPT,), jnp.float32),
            pltpu.VMEM((GP,), jnp.float32),
            pltpu.VMEM((GP,), jnp.float32),
            pltpu.VMEM((GP,), jnp.float32),
            pltpu.VMEM((16,), jnp.float32),
            pltpu.VMEM((GP,), jnp.float32),
            pltpu.VMEM_SHARED((GP,), jnp.float32),
        ],
        name="sc_pool",
        compiler_params=pltpu.CompilerParams(use_tc_tiling_on_sc=False),
    )(q2, b3, cnt2, blb)


def _tc_g1_body(deg2, h0, w1, glo, ghi, dinv):
    deg = deg2[0, :] + deg2[1, :] + 1.0
    dv = lax.rsqrt(deg)
    t = jnp.dot(h0[...], w1[...], preferred_element_type=jnp.float32,
                precision=lax.Precision.HIGHEST)
    t = t * dv[:, None]
    glo[...] = t[:, :HALF]
    ghi[...] = t[:, HALF:]
    dinv[...] = dv


def _tc_g1(deg2, h0, W1):
    grid = (NP // BLK,)
    return pl.pallas_call(
        _tc_g1_body,
        grid=grid,
        in_specs=[
            pl.BlockSpec((NCORE, BLK), lambda j: (0, j)),
            pl.BlockSpec((BLK, D), lambda j: (j, 0)),
            pl.BlockSpec((D, D), lambda j: (0, 0)),
        ],
        out_specs=[
            pl.BlockSpec((BLK, HALF), lambda j: (j, 0)),
            pl.BlockSpec((BLK, HALF), lambda j: (j, 0)),
            pl.BlockSpec((BLK,), lambda j: (j,)),
        ],
        out_shape=[
            jax.ShapeDtypeStruct((NP, HALF), jnp.float32),
            jax.ShapeDtypeStruct((NP, HALF), jnp.float32),
            jax.ShapeDtypeStruct((NP,), jnp.float32),
        ],
        name="tc_g1",
    )(deg2, h0, W1)


def _tc_g2_body(alo, ahi, glo, ghi, dinv, b1, w2, olo, ohi):
    dv = dinv[...]
    h = jnp.concatenate([alo[...] + glo[...], ahi[...] + ghi[...]], axis=1)
    h = jnp.maximum(h * dv[:, None] + b1[...], 0.0)
    t = jnp.dot(h, w2[...], preferred_element_type=jnp.float32,
                precision=lax.Precision.HIGHEST)
    t = t * dv[:, None]
    olo[...] = t[:, :HALF]
    ohi[...] = t[:, HALF:]


def _tc_g2(alo, ahi, glo, ghi, dinv, b1r, W2):
    grid = (NP // BLK,)
    half_spec = pl.BlockSpec((BLK, HALF), lambda j: (j, 0))
    return pl.pallas_call(
        _tc_g2_body,
        grid=grid,
        in_specs=[
            half_spec, half_spec, half_spec, half_spec,
            pl.BlockSpec((BLK,), lambda j: (j,)),
            pl.BlockSpec((1, D), lambda j: (0, 0)),
            pl.BlockSpec((D, D), lambda j: (0, 0)),
        ],
        out_specs=[half_spec, half_spec],
        out_shape=[
            jax.ShapeDtypeStruct((NP, HALF), jnp.float32),
            jax.ShapeDtypeStruct((NP, HALF), jnp.float32),
        ],
        name="tc_g2",
    )(alo, ahi, glo, ghi, dinv, b1r, W2)


def _tc_q_body(alo, ahi, glo, ghi, dinv, b2, wl, q0, q1):
    dv = dinv[...]
    h = jnp.concatenate([alo[...] + glo[...], ahi[...] + ghi[...]], axis=1)
    h = jnp.maximum(h * dv[:, None] + b2[...], 0.0)
    w = wl[...]
    q0[...] = jnp.sum(h * w[:, 0][None, :], axis=1)
    q1[...] = jnp.sum(h * w[:, 1][None, :], axis=1)


def _tc_q(alo, ahi, glo, ghi, dinv, b2r, Wl):
    grid = (NP // BLK,)
    half_spec = pl.BlockSpec((BLK, HALF), lambda j: (j, 0))
    return pl.pallas_call(
        _tc_q_body,
        grid=grid,
        in_specs=[
            half_spec, half_spec, half_spec, half_spec,
            pl.BlockSpec((BLK,), lambda j: (j,)),
            pl.BlockSpec((1, D), lambda j: (0, 0)),
            pl.BlockSpec((D, NCL), lambda j: (0, 0)),
        ],
        out_specs=[
            pl.BlockSpec((BLK,), lambda j: (j,)),
            pl.BlockSpec((BLK,), lambda j: (j,)),
        ],
        out_shape=[
            jax.ShapeDtypeStruct((NP,), jnp.float32),
            jax.ShapeDtypeStruct((NP,), jnp.float32),
        ],
        name="tc_q",
    )(alo, ahi, glo, ghi, dinv, b2r, Wl)


@jax.jit
def _run(x, edge_index, batch, emb, W1, b1, W2, b2, Wl, bl):
    src = edge_index[0].astype(jnp.int32)
    dst = edge_index[1].astype(jnp.int32)
    xi = x.astype(jnp.int32)
    bi = batch.astype(jnp.int32)

    pad_e = jnp.full((EP - E,), DUM, jnp.int32)
    src_p = jnp.concatenate([src, pad_e])
    dst_p = jnp.concatenate([dst, pad_e])
    src3 = src_p.reshape(NTILE, EC_T, CH)
    dst3 = dst_p.reshape(NTILE, EC_T, CH)
    dstA = dst_p.reshape(NTILE * NCORE, EC_T // 2, CH)
    x3 = jnp.concatenate([xi, jnp.zeros((NP - N,), jnp.int32)]).reshape(
        NTILE * NCORE, NC_W, CH)
    batch_p = jnp.concatenate([bi, jnp.full((NP - N,), G, jnp.int32)])
    b3 = batch_p.reshape(NTILE, NC_T, CH)
    bA = batch_p.reshape(NTILE * NCORE, NC_W, CH)

    z1 = jnp.zeros((NP,), jnp.float32)
    z2 = jnp.zeros((NP, HALF), jnp.float32)
    b1r = b1.reshape(1, D)
    b2r = b2.reshape(1, D)
    blb = jnp.broadcast_to(bl[:, None], (NCL, 16))

    h0, deg2, cnt2 = _sc_prep(emb, x3, dstA, bA, z1)
    g1lo, g1hi, dinv = _tc_g1(deg2, h0, W1)
    a1lo, a1hi = _sc_prop(g1lo, g1hi, src3, dst3, z2)
    g2lo, g2hi = _tc_g2(a1lo, a1hi, g1lo, g1hi, dinv, b1r, W2)
    a2lo, a2hi = _sc_prop(g2lo, g2hi, src3, dst3, z2)
    q0, q1 = _tc_q(a2lo, a2hi, g2lo, g2hi, dinv, b2r, Wl)
    q2 = jnp.stack([q0, q1])
    res = _sc_pool(q2, b3, cnt2, blb)
    return res[:, :G].T


def kernel(x, edge_index, batch, emb, W1, b1, W2, b2, Wl, bl):
    return _run(x, edge_index, batch, emb, W1, b1, W2, b2, Wl, bl)


# speedup vs baseline: 26.7170x; 1.1918x over previous
import functools

import jax
import jax.numpy as jnp
from jax import lax
from jax.experimental import pallas as pl
from jax.experimental.pallas import tpu as pltpu
from jax.experimental.pallas import tpu_sc as plsc

N = 50000
E = 800000
G = 512
VOCAB = 100000
D = 64
NCL = 2

NTILE = 16
NCORE = 2
CH = 128
EC_T = 392
EP = NTILE * EC_T * CH
NP = 53248
NC_T = NP // CH // NTILE
NC_W = NP // CH // (NTILE * NCORE)
NPT = NP // NTILE
GP = 528
DUM = N
BLK = 1024
HALF = D // 2


def _sc_prep_body(emb_h, x3_h, dstA_h, bA_h, z1_h,
                  h0_h, deg2_h, cnt2_h,
                  xv, rows, dstv, bv, ones_v, zg_v, deg_sh, cnt_sh, gsem):
    c = lax.axis_index("c")
    s = lax.axis_index("s")
    wid = c * NTILE + s

    for i in range(CH // 16):
        ones_v[pl.ds(i * 16, 16)] = jnp.full((16,), 1.0, jnp.float32)

    pltpu.sync_copy(z1_h.at[pl.ds(s * NPT, NPT)], deg_sh.at[pl.ds(s * NPT, NPT)])

    @pl.when(s == 0)
    def _():
        for i in range(GP // 16):
            zg_v[pl.ds(i * 16, 16)] = jnp.zeros((16,), jnp.float32)
        pltpu.sync_copy(zg_v, cnt_sh)

    pltpu.sync_copy(x3_h.at[wid], xv)

    def emb_body(k, _):
        pltpu.async_copy(emb_h.at[xv.at[k]], rows, gsem).wait()
        pltpu.sync_copy(rows, h0_h.at[pl.ds(wid * NC_W * CH + k * CH, CH)])
        return 0

    lax.fori_loop(0, NC_W, emb_body, 0)

    plsc.subcore_barrier()

    pltpu.sync_copy(dstA_h.at[wid], dstv)

    def deg_body(k, _):
        pltpu.sync_copy(ones_v, deg_sh.at[dstv.at[k]], add=True)
        return 0

    lax.fori_loop(0, EC_T // 2, deg_body, 0)

    pltpu.sync_copy(bA_h.at[wid], bv)

    def cnt_body(k, _):
        pltpu.sync_copy(ones_v, cnt_sh.at[bv.at[k]], add=True)
        return 0

    lax.fori_loop(0, NC_W, cnt_body, 0)

    plsc.subcore_barrier()

    pltpu.sync_copy(deg_sh.at[pl.ds(s * NPT, NPT)],
                    deg2_h.at[c, pl.ds(s * NPT, NPT)])

    @pl.when(s == 0)
    def _():
        pltpu.sync_copy(cnt_sh, cnt2_h.at[c])


def _sc_prep(emb, x3, dstA, bA, z1):
    mesh = plsc.VectorSubcoreMesh(core_axis_name="c", subcore_axis_name="s")
    return pl.kernel(
        _sc_prep_body,
        out_type=[
            jax.ShapeDtypeStruct((NP, D), jnp.float32),
            jax.ShapeDtypeStruct((NCORE, NP), jnp.float32),
            jax.ShapeDtypeStruct((NCORE, GP), jnp.float32),
        ],
        mesh=mesh,
        scratch_types=[
            pltpu.VMEM((NC_W, CH), jnp.int32),
            pltpu.VMEM((CH, D), jnp.float32),
            pltpu.VMEM((EC_T // 2, CH), jnp.int32),
            pltpu.VMEM((NC_W, CH), jnp.int32),
            pltpu.VMEM((CH,), jnp.float32),
            pltpu.VMEM((GP,), jnp.float32),
            pltpu.VMEM_SHARED((NP,), jnp.float32),
            pltpu.VMEM_SHARED((GP,), jnp.float32),
            pltpu.SemaphoreType.DMA,
        ],
        name="sc_prep",
        compiler_params=pltpu.CompilerParams(use_tc_tiling_on_sc=False),
    )(emb, x3, dstA, bA, z1)


IB = 8
NG = EC_T // IB
NRB = 4


def _sc_prop_body(glo_h, ghi_h, src3_h, dst3_h, z2_h,
                  alo_h, ahi_h,
                  sbuf, dbuf, r0, r1, r2, r3, acc_sh,
                  gs0, gs1, gs2, gs3, isem):
    c = lax.axis_index("c")
    s = lax.axis_index("s")
    rbufs = (r0, r1, r2, r3)
    gsems = (gs0, gs1, gs2, gs3)

    pltpu.sync_copy(z2_h.at[pl.ds(s * NPT, NPT)], acc_sh.at[pl.ds(s * NPT, NPT)])
    plsc.subcore_barrier()

    def run(g_h, a_h):
        for g0 in range(2):
            pltpu.async_copy(src3_h.at[s, pl.ds(g0 * IB, IB)], sbuf.at[g0], isem)
            pltpu.async_copy(dst3_h.at[s, pl.ds(g0 * IB, IB)], dbuf.at[g0], isem)
        for g0 in range(2):
            pltpu.make_async_copy(src3_h.at[s, pl.ds(g0 * IB, IB)],
                                  sbuf.at[g0], isem).wait()
            pltpu.make_async_copy(dst3_h.at[s, pl.ds(g0 * IB, IB)],
                                  dbuf.at[g0], isem).wait()

        for b in range(NRB):
            pltpu.async_copy(g_h.at[sbuf.at[0, b]], rbufs[b], gsems[b])

        def group(g, _):
            buf = lax.rem(g, 3)

            @pl.when((g >= 1) & (g + 1 < NG))
            def _():
                nb = lax.rem(g + 1, 3)
                pltpu.make_async_copy(src3_h.at[s, pl.ds((g + 1) * IB, IB)],
                                      sbuf.at[nb], isem).wait()
                pltpu.make_async_copy(dst3_h.at[s, pl.ds((g + 1) * IB, IB)],
                                      dbuf.at[nb], isem).wait()

            @pl.when(g + 2 < NG)
            def _():
                nb2 = lax.rem(g + 2, 3)
                pltpu.async_copy(src3_h.at[s, pl.ds((g + 2) * IB, IB)],
                                 sbuf.at[nb2], isem)
                pltpu.async_copy(dst3_h.at[s, pl.ds((g + 2) * IB, IB)],
                                 dbuf.at[nb2], isem)

            for b in range(IB):
                rb = rbufs[b % NRB]
                sem = gsems[b % NRB]
                pltpu.make_async_copy(g_h.at[sbuf.at[buf, b]], rb, sem).wait()
                pltpu.sync_copy(rb, acc_sh.at[dbuf.at[buf, b]], add=True)
                kk = b + NRB
                if kk < IB:
                    pltpu.async_copy(g_h.at[sbuf.at[buf, kk]], rb, sem)
                else:
                    @pl.when(g + 1 < NG)
                    def _(rb=rb, sem=sem, kk=kk):
                        nb = lax.rem(g + 1, 3)
                        pltpu.async_copy(g_h.at[sbuf.at[nb, kk - IB]], rb, sem)
            return 0

        lax.fori_loop(0, NG, group, 0)
        plsc.subcore_barrier()
        pltpu.sync_copy(acc_sh.at[pl.ds(s * NPT, NPT)],
                        a_h.at[pl.ds(s * NPT, NPT)])

    @pl.when(c == 0)
    def _():
        run(glo_h, alo_h)

    @pl.when(c == 1)
    def _():
        run(ghi_h, ahi_h)


def _sc_prop(g_lo, g_hi, src3, dst3, z2):
    mesh = plsc.VectorSubcoreMesh(core_axis_name="c", subcore_axis_name="s")
    return pl.kernel(
        _sc_prop_body,
        out_type=[
            jax.ShapeDtypeStruct((NP, HALF), jnp.float32),
            jax.ShapeDtypeStruct((NP, HALF), jnp.float32),
        ],
        mesh=mesh,
        scratch_types=[
            pltpu.VMEM((3, IB, CH), jnp.int32),
            pltpu.VMEM((3, IB, CH), jnp.int32),
            pltpu.VMEM((CH, HALF), jnp.float32),
            pltpu.VMEM((CH, HALF), jnp.float32),
            pltpu.VMEM((CH, HALF), jnp.float32),
            pltpu.VMEM((CH, HALF), jnp.float32),
            pltpu.VMEM_SHARED((NP, HALF), jnp.float32),
            pltpu.SemaphoreType.DMA,
            pltpu.SemaphoreType.DMA,
            pltpu.SemaphoreType.DMA,
            pltpu.SemaphoreType.DMA,
            pltpu.SemaphoreType.DMA,
        ],
        name="sc_prop",
        compiler_params=pltpu.CompilerParams(use_tc_tiling_on_sc=False),
    )(g_lo, g_hi, src3, dst3, z2)


def _sc_pool_body(q2_h, b3_h, cnt2_h, blb_h, res_h,
                  bv, qv, sums_v, ca, cb, blv, res_v, sums_sh):
    c = lax.axis_index("c")
    s = lax.axis_index("s")

    @pl.when(s == 0)
    def _():
        for i in range(GP // 16):
            res_v[pl.ds(i * 16, 16)] = jnp.zeros((16,), jnp.float32)
        pltpu.sync_copy(res_v, sums_sh)

    pltpu.sync_copy(q2_h.at[c, pl.ds(s * NPT, NPT)], qv)
    pltpu.sync_copy(b3_h.at[s], bv)
    plsc.subcore_barrier()

    def body(k, _):
        pltpu.sync_copy(qv.at[pl.ds(k * CH, CH)], sums_sh.at[bv.at[k]], add=True)
        return 0

    lax.fori_loop(0, NC_T, body, 0)
    plsc.subcore_barrier()

    @pl.when(s == 0)
    def _():
        pltpu.sync_copy(sums_sh, sums_v)
        pltpu.sync_copy(cnt2_h.at[0], ca)
        pltpu.sync_copy(cnt2_h.at[1], cb)
        pltpu.sync_copy(blb_h.at[c], blv)
        bvec = blv[...]
        for i in range(GP // 16):
            d = pl.ds(i * 16, 16)
            cv = ca[d] + cb[d]
            res_v[d] = sums_v[d] / jnp.maximum(cv, 1.0) + bvec
        pltpu.sync_copy(res_v, res_h.at[c])


def _sc_pool(q2, b3, cnt2, blb):
    mesh = plsc.VectorSubcoreMesh(core_axis_name="c", subcore_axis_name="s")
    return pl.kernel(
        _sc_pool_body,
        out_type=jax.ShapeDtypeStruct((NCORE, GP), jnp.float32),
        mesh=mesh,
        scratch_types=[
            pltpu.VMEM((NC_T, CH), jnp.int32),
            pltpu.VMEM((NPT,), jnp.float32),
            pltpu.VMEM((GP,), jnp.float32),
            pltpu.VMEM((GP,), jnp.float32),
            pltpu.VMEM((GP,), jnp.float32),
            pltpu.VMEM((16,), jnp.float32),
            pltpu.VMEM((GP,), jnp.float32),
            pltpu.VMEM_SHARED((GP,), jnp.float32),
        ],
        name="sc_pool",
        compiler_params=pltpu.CompilerParams(use_tc_tiling_on_sc=False),
    )(q2, b3, cnt2, blb)


def _tc_g1_body(deg2, h0, w1, glo, ghi, dinv):
    deg = deg2[0, :] + deg2[1, :] + 1.0
    dv = lax.rsqrt(deg)
    t = jnp.dot(h0[...], w1[...], preferred_element_type=jnp.float32,
                precision=lax.Precision.HIGHEST)
    t = t * dv[:, None]
    glo[...] = t[:, :HALF]
    ghi[...] = t[:, HALF:]
    dinv[...] = dv


def _tc_g1(deg2, h0, W1):
    grid = (NP // BLK,)
    return pl.pallas_call(
        _tc_g1_body,
        grid=grid,
        in_specs=[
            pl.BlockSpec((NCORE, BLK), lambda j: (0, j)),
            pl.BlockSpec((BLK, D), lambda j: (j, 0)),
            pl.BlockSpec((D, D), lambda j: (0, 0)),
        ],
        out_specs=[
            pl.BlockSpec((BLK, HALF), lambda j: (j, 0)),
            pl.BlockSpec((BLK, HALF), lambda j: (j, 0)),
            pl.BlockSpec((BLK,), lambda j: (j,)),
        ],
        out_shape=[
            jax.ShapeDtypeStruct((NP, HALF), jnp.float32),
            jax.ShapeDtypeStruct((NP, HALF), jnp.float32),
            jax.ShapeDtypeStruct((NP,), jnp.float32),
        ],
        name="tc_g1",
    )(deg2, h0, W1)


def _tc_g2_body(alo, ahi, glo, ghi, dinv, b1, w2, olo, ohi):
    dv = dinv[...]
    h = jnp.concatenate([alo[...] + glo[...], ahi[...] + ghi[...]], axis=1)
    h = jnp.maximum(h * dv[:, None] + b1[...], 0.0)
    t = jnp.dot(h, w2[...], preferred_element_type=jnp.float32,
                precision=lax.Precision.HIGHEST)
    t = t * dv[:, None]
    olo[...] = t[:, :HALF]
    ohi[...] = t[:, HALF:]


def _tc_g2(alo, ahi, glo, ghi, dinv, b1r, W2):
    grid = (NP // BLK,)
    half_spec = pl.BlockSpec((BLK, HALF), lambda j: (j, 0))
    return pl.pallas_call(
        _tc_g2_body,
        grid=grid,
        in_specs=[
            half_spec, half_spec, half_spec, half_spec,
            pl.BlockSpec((BLK,), lambda j: (j,)),
            pl.BlockSpec((1, D), lambda j: (0, 0)),
            pl.BlockSpec((D, D), lambda j: (0, 0)),
        ],
        out_specs=[half_spec, half_spec],
        out_shape=[
            jax.ShapeDtypeStruct((NP, HALF), jnp.float32),
            jax.ShapeDtypeStruct((NP, HALF), jnp.float32),
        ],
        name="tc_g2",
    )(alo, ahi, glo, ghi, dinv, b1r, W2)


def _tc_q_body(alo, ahi, glo, ghi, dinv, b2, wl, q0, q1):
    dv = dinv[...]
    h = jnp.concatenate([alo[...] + glo[...], ahi[...] + ghi[...]], axis=1)
    h = jnp.maximum(h * dv[:, None] + b2[...], 0.0)
    w = wl[...]
    q0[...] = jnp.sum(h * w[:, 0][None, :], axis=1)
    q1[...] = jnp.sum(h * w[:, 1][None, :], axis=1)


def _tc_q(alo, ahi, glo, ghi, dinv, b2r, Wl):
    grid = (NP // BLK,)
    half_spec = pl.BlockSpec((BLK, HALF), lambda j: (j, 0))
    return pl.pallas_call(
        _tc_q_body,
        grid=grid,
        in_specs=[
            half_spec, half_spec, half_spec, half_spec,
            pl.BlockSpec((BLK,), lambda j: (j,)),
            pl.BlockSpec((1, D), lambda j: (0, 0)),
            pl.BlockSpec((D, NCL), lambda j: (0, 0)),
        ],
        out_specs=[
            pl.BlockSpec((BLK,), lambda j: (j,)),
            pl.BlockSpec((BLK,), lambda j: (j,)),
        ],
        out_shape=[
            jax.ShapeDtypeStruct((NP,), jnp.float32),
            jax.ShapeDtypeStruct((NP,), jnp.float32),
        ],
        name="tc_q",
    )(alo, ahi, glo, ghi, dinv, b2r, Wl)


@jax.jit
def _run(x, edge_index, batch, emb, W1, b1, W2, b2, Wl, bl):
    src = edge_index[0].astype(jnp.int32)
    dst = edge_index[1].astype(jnp.int32)
    xi = x.astype(jnp.int32)
    bi = batch.astype(jnp.int32)

    pad_e = jnp.full((EP - E,), DUM, jnp.int32)
    src_p = jnp.concatenate([src, pad_e])
    dst_p = jnp.concatenate([dst, pad_e])
    src3 = src_p.reshape(NTILE, EC_T, CH)
    dst3 = dst_p.reshape(NTILE, EC_T, CH)
    dstA = dst_p.reshape(NTILE * NCORE, EC_T // 2, CH)
    x3 = jnp.concatenate([xi, jnp.zeros((NP - N,), jnp.int32)]).reshape(
        NTILE * NCORE, NC_W, CH)
    batch_p = jnp.concatenate([bi, jnp.full((NP - N,), G, jnp.int32)])
    b3 = batch_p.reshape(NTILE, NC_T, CH)
    bA = batch_p.reshape(NTILE * NCORE, NC_W, CH)

    z1 = jnp.zeros((NP,), jnp.float32)
    z2 = jnp.zeros((NP, HALF), jnp.float32)
    b1r = b1.reshape(1, D)
    b2r = b2.reshape(1, D)
    blb = jnp.broadcast_to(bl[:, None], (NCL, 16))

    h0, deg2, cnt2 = _sc_prep(emb, x3, dstA, bA, z1)
    g1lo, g1hi, dinv = _tc_g1(deg2, h0, W1)
    a1lo, a1hi = _sc_prop(g1lo, g1hi, src3, dst3, z2)
    g2lo, g2hi = _tc_g2(a1lo, a1hi, g1lo, g1hi, dinv, b1r, W2)
    a2lo, a2hi = _sc_prop(g2lo, g2hi, src3, dst3, z2)
    q0, q1 = _tc_q(a2lo, a2hi, g2lo, g2hi, dinv, b2r, Wl)
    q2 = jnp.stack([q0, q1])
    res = _sc_pool(q2, b3, cnt2, blb)
    return res[:, :G].T


def kernel(x, edge_index, batch, emb, W1, b1, W2, b2, Wl, bl):
    return _run(x, edge_index, batch, emb, W1, b1, W2, b2, Wl, bl)


# speedup vs baseline: 27.5609x; 1.0316x over previous
import functools

import jax
import jax.numpy as jnp
from jax import lax
from jax.experimental import pallas as pl
from jax.experimental.pallas import tpu as pltpu
from jax.experimental.pallas import tpu_sc as plsc

N = 50000
E = 800000
G = 512
VOCAB = 100000
D = 64
NCL = 2

NTILE = 16
NCORE = 2
CH = 128
EC_T = 392
EP = NTILE * EC_T * CH
NP = 53248
NC_T = NP // CH // NTILE
NC_W = NP // CH // (NTILE * NCORE)
NPT = NP // NTILE
GP = 528
DUM = N
BLK = 1024
HALF = D // 2


EMB_BATCHES = (4, 4, 4, 1)


def _sc_prep_body(emb_h, x3_h, dstA_h, bA_h, z1_h,
                  h0_h, deg2_h, cnt2_h,
                  xv, rows, dstv, bv, ones_v, zg_v, deg_sh, cnt_sh,
                  dsem, gsem, wsem, hsem, csem):
    c = lax.axis_index("c")
    s = lax.axis_index("s")
    wid = c * NTILE + s

    pltpu.async_copy(dstA_h.at[wid], dstv, dsem)
    pltpu.sync_copy(x3_h.at[wid], xv)
    pltpu.sync_copy(bA_h.at[wid], bv)

    for i in range(CH // 16):
        ones_v[pl.ds(i * 16, 16)] = jnp.full((16,), 1.0, jnp.float32)

    pltpu.sync_copy(z1_h.at[pl.ds(s * NPT, NPT)], deg_sh.at[pl.ds(s * NPT, NPT)])

    @pl.when(s == 0)
    def _():
        for i in range(GP // 16):
            zg_v[pl.ds(i * 16, 16)] = jnp.zeros((16,), jnp.float32)
        pltpu.sync_copy(zg_v, cnt_sh)

    plsc.subcore_barrier()

    pltpu.make_async_copy(dstA_h.at[wid], dstv, dsem).wait()

    def deg_fire(k, _):
        pltpu.async_copy(ones_v, deg_sh.at[dstv.at[k]], hsem, add=True)
        return 0

    lax.fori_loop(0, EC_T // 2, deg_fire, 0)

    def cnt_fire(k, _):
        pltpu.async_copy(ones_v, cnt_sh.at[bv.at[k]], csem, add=True)
        return 0

    lax.fori_loop(0, NC_W, cnt_fire, 0)

    base = wid * NC_W * CH
    starts = [sum(EMB_BATCHES[:i]) for i in range(len(EMB_BATCHES))]

    def fire_gathers(bi):
        for j in range(EMB_BATCHES[bi]):
            k = starts[bi] + j
            pltpu.async_copy(emb_h.at[xv.at[k]], rows.at[k % 8], gsem)

    def drain_writebacks(bi):
        for j in range(EMB_BATCHES[bi]):
            k = starts[bi] + j
            pltpu.make_async_copy(rows.at[k % 8],
                                  h0_h.at[pl.ds(base + k * CH, CH)],
                                  wsem).wait()

    fire_gathers(0)
    for bi in range(len(EMB_BATCHES)):
        if bi >= 1:
            drain_writebacks(bi - 1)
        if bi + 1 < len(EMB_BATCHES):
            fire_gathers(bi + 1)
        for j in range(EMB_BATCHES[bi]):
            k = starts[bi] + j
            pltpu.make_async_copy(emb_h.at[xv.at[k]], rows.at[k % 8],
                                  gsem).wait()
        for j in range(EMB_BATCHES[bi]):
            k = starts[bi] + j
            pltpu.async_copy(rows.at[k % 8],
                             h0_h.at[pl.ds(base + k * CH, CH)], wsem)
    drain_writebacks(len(EMB_BATCHES) - 1)

    def deg_drain(k, _):
        pltpu.make_async_copy(ones_v, deg_sh.at[dstv.at[k]], hsem).wait()
        return 0

    lax.fori_loop(0, EC_T // 2, deg_drain, 0)

    def cnt_drain(k, _):
        pltpu.make_async_copy(ones_v, cnt_sh.at[bv.at[k]], csem).wait()
        return 0

    lax.fori_loop(0, NC_W, cnt_drain, 0)

    plsc.subcore_barrier()

    pltpu.sync_copy(deg_sh.at[pl.ds(s * NPT, NPT)],
                    deg2_h.at[c, pl.ds(s * NPT, NPT)])

    @pl.when(s == 0)
    def _():
        pltpu.sync_copy(cnt_sh, cnt2_h.at[c])


def _sc_prep(emb, x3, dstA, bA, z1):
    mesh = plsc.VectorSubcoreMesh(core_axis_name="c", subcore_axis_name="s")
    return pl.kernel(
        _sc_prep_body,
        out_type=[
            jax.ShapeDtypeStruct((NP, D), jnp.float32),
            jax.ShapeDtypeStruct((NCORE, NP), jnp.float32),
            jax.ShapeDtypeStruct((NCORE, GP), jnp.float32),
        ],
        mesh=mesh,
        scratch_types=[
            pltpu.VMEM((NC_W, CH), jnp.int32),
            pltpu.VMEM((8, CH, D), jnp.float32),
            pltpu.VMEM((EC_T // 2, CH), jnp.int32),
            pltpu.VMEM((NC_W, CH), jnp.int32),
            pltpu.VMEM((CH,), jnp.float32),
            pltpu.VMEM((GP,), jnp.float32),
            pltpu.VMEM_SHARED((NP,), jnp.float32),
            pltpu.VMEM_SHARED((GP,), jnp.float32),
            pltpu.SemaphoreType.DMA,
            pltpu.SemaphoreType.DMA,
            pltpu.SemaphoreType.DMA,
            pltpu.SemaphoreType.DMA,
            pltpu.SemaphoreType.DMA,
        ],
        name="sc_prep",
        compiler_params=pltpu.CompilerParams(use_tc_tiling_on_sc=False),
    )(emb, x3, dstA, bA, z1)


IB = 8
NG = EC_T // IB
NRB = 4


def _sc_prop_body(glo_h, ghi_h, src3_h, dst3_h, z2c_h,
                  alo_h, ahi_h,
                  sbuf, dbuf, r0, r1, r2, r3, acc_sh,
                  gs0, gs1, gs2, gs3, isem):
    c = lax.axis_index("c")
    s = lax.axis_index("s")
    rbufs = (r0, r1, r2, r3)
    gsems = (gs0, gs1, gs2, gs3)

    pltpu.sync_copy(z2c_h, r0)

    def zero_body(i, _):
        pltpu.sync_copy(r0, acc_sh.at[pl.ds(s * NPT + i * CH, CH)])
        return 0

    lax.fori_loop(0, NPT // CH, zero_body, 0)
    plsc.subcore_barrier()

    def run(g_h, a_h):
        for g0 in range(2):
            pltpu.async_copy(src3_h.at[s, pl.ds(g0 * IB, IB)], sbuf.at[g0], isem)
            pltpu.async_copy(dst3_h.at[s, pl.ds(g0 * IB, IB)], dbuf.at[g0], isem)
        for g0 in range(2):
            pltpu.make_async_copy(src3_h.at[s, pl.ds(g0 * IB, IB)],
                                  sbuf.at[g0], isem).wait()
            pltpu.make_async_copy(dst3_h.at[s, pl.ds(g0 * IB, IB)],
                                  dbuf.at[g0], isem).wait()

        for b in range(NRB):
            pltpu.async_copy(g_h.at[sbuf.at[0, b]], rbufs[b], gsems[b])

        def group(g, _):
            buf = lax.rem(g, 3)

            @pl.when((g >= 1) & (g + 1 < NG))
            def _():
                nb = lax.rem(g + 1, 3)
                pltpu.make_async_copy(src3_h.at[s, pl.ds((g + 1) * IB, IB)],
                                      sbuf.at[nb], isem).wait()
                pltpu.make_async_copy(dst3_h.at[s, pl.ds((g + 1) * IB, IB)],
                                      dbuf.at[nb], isem).wait()

            @pl.when(g + 2 < NG)
            def _():
                nb2 = lax.rem(g + 2, 3)
                pltpu.async_copy(src3_h.at[s, pl.ds((g + 2) * IB, IB)],
                                 sbuf.at[nb2], isem)
                pltpu.async_copy(dst3_h.at[s, pl.ds((g + 2) * IB, IB)],
                                 dbuf.at[nb2], isem)

            for b in range(IB):
                rb = rbufs[b % NRB]
                sem = gsems[b % NRB]
                pltpu.make_async_copy(g_h.at[sbuf.at[buf, b]], rb, sem).wait()
                pltpu.sync_copy(rb, acc_sh.at[dbuf.at[buf, b]], add=True)
                kk = b + NRB
                if kk < IB:
                    pltpu.async_copy(g_h.at[sbuf.at[buf, kk]], rb, sem)
                else:
                    @pl.when(g + 1 < NG)
                    def _(rb=rb, sem=sem, kk=kk):
                        nb = lax.rem(g + 1, 3)
                        pltpu.async_copy(g_h.at[sbuf.at[nb, kk - IB]], rb, sem)
            return 0

        lax.fori_loop(0, NG, group, 0)
        plsc.subcore_barrier()
        pltpu.sync_copy(acc_sh.at[pl.ds(s * NPT, NPT)],
                        a_h.at[pl.ds(s * NPT, NPT)])

    @pl.when(c == 0)
    def _():
        run(glo_h, alo_h)

    @pl.when(c == 1)
    def _():
        run(ghi_h, ahi_h)


def _sc_prop(g_lo, g_hi, src3, dst3, z2c):
    mesh = plsc.VectorSubcoreMesh(core_axis_name="c", subcore_axis_name="s")
    return pl.kernel(
        _sc_prop_body,
        out_type=[
            jax.ShapeDtypeStruct((NP, HALF), jnp.float32),
            jax.ShapeDtypeStruct((NP, HALF), jnp.float32),
        ],
        mesh=mesh,
        scratch_types=[
            pltpu.VMEM((3, IB, CH), jnp.int32),
            pltpu.VMEM((3, IB, CH), jnp.int32),
            pltpu.VMEM((CH, HALF), jnp.float32),
            pltpu.VMEM((CH, HALF), jnp.float32),
            pltpu.VMEM((CH, HALF), jnp.float32),
            pltpu.VMEM((CH, HALF), jnp.float32),
            pltpu.VMEM_SHARED((NP, HALF), jnp.float32),
            pltpu.SemaphoreType.DMA,
            pltpu.SemaphoreType.DMA,
            pltpu.SemaphoreType.DMA,
            pltpu.SemaphoreType.DMA,
            pltpu.SemaphoreType.DMA,
        ],
        name="sc_prop",
        compiler_params=pltpu.CompilerParams(use_tc_tiling_on_sc=False),
    )(g_lo, g_hi, src3, dst3, z2c)


def _sc_pool_body(q2_h, b3_h, cnt2_h, blb_h, res_h,
                  bv, qv, sums_v, ca, cb, blv, res_v, sums_sh):
    c = lax.axis_index("c")
    s = lax.axis_index("s")

    @pl.when(s == 0)
    def _():
        for i in range(GP // 16):
            res_v[pl.ds(i * 16, 16)] = jnp.zeros((16,), jnp.float32)
        pltpu.sync_copy(res_v, sums_sh)

    pltpu.sync_copy(q2_h.at[c, pl.ds(s * NPT, NPT)], qv)
    pltpu.sync_copy(b3_h.at[s], bv)
    plsc.subcore_barrier()

    def body(k, _):
        pltpu.sync_copy(qv.at[pl.ds(k * CH, CH)], sums_sh.at[bv.at[k]], add=True)
        return 0

    lax.fori_loop(0, NC_T, body, 0)
    plsc.subcore_barrier()

    @pl.when(s == 0)
    def _():
        pltpu.sync_copy(sums_sh, sums_v)
        pltpu.sync_copy(cnt2_h.at[0], ca)
        pltpu.sync_copy(cnt2_h.at[1], cb)
        pltpu.sync_copy(blb_h.at[c], blv)
        bvec = blv[...]
        for i in range(GP // 16):
            d = pl.ds(i * 16, 16)
            cv = ca[d] + cb[d]
            res_v[d] = sums_v[d] / jnp.maximum(cv, 1.0) + bvec
        pltpu.sync_copy(res_v, res_h.at[c])


def _sc_pool(q2, b3, cnt2, blb):
    mesh = plsc.VectorSubcoreMesh(core_axis_name="c", subcore_axis_name="s")
    return pl.kernel(
        _sc_pool_body,
        out_type=jax.ShapeDtypeStruct((NCORE, GP), jnp.float32),
        mesh=mesh,
        scratch_types=[
            pltpu.VMEM((NC_T, CH), jnp.int32),
            pltpu.VMEM((NPT,), jnp.float32),
            pltpu.VMEM((GP,), jnp.float32),
            pltpu.VMEM((GP,), jnp.float32),
            pltpu.VMEM((GP,), jnp.float32),
            pltpu.VMEM((16,), jnp.float32),
            pltpu.VMEM((GP,), jnp.float32),
            pltpu.VMEM_SHARED((GP,), jnp.float32),
        ],
        name="sc_pool",
        compiler_params=pltpu.CompilerParams(use_tc_tiling_on_sc=False),
    )(q2, b3, cnt2, blb)


def _tc_g1_body(deg2, h0, w1, glo, ghi, dinv):
    deg = deg2[0, :] + deg2[1, :] + 1.0
    dv = lax.rsqrt(deg)
    t = jnp.dot(h0[...], w1[...], preferred_element_type=jnp.float32,
                precision=lax.Precision.HIGHEST)
    t = t * dv[:, None]
    glo[...] = t[:, :HALF]
    ghi[...] = t[:, HALF:]
    dinv[...] = dv


def _tc_g1(deg2, h0, W1):
    grid = (NP // BLK,)
    return pl.pallas_call(
        _tc_g1_body,
        grid=grid,
        in_specs=[
            pl.BlockSpec((NCORE, BLK), lambda j: (0, j)),
            pl.BlockSpec((BLK, D), lambda j: (j, 0)),
            pl.BlockSpec((D, D), lambda j: (0, 0)),
        ],
        out_specs=[
            pl.BlockSpec((BLK, HALF), lambda j: (j, 0)),
            pl.BlockSpec((BLK, HALF), lambda j: (j, 0)),
            pl.BlockSpec((BLK,), lambda j: (j,)),
        ],
        out_shape=[
            jax.ShapeDtypeStruct((NP, HALF), jnp.float32),
            jax.ShapeDtypeStruct((NP, HALF), jnp.float32),
            jax.ShapeDtypeStruct((NP,), jnp.float32),
        ],
        name="tc_g1",
    )(deg2, h0, W1)


def _tc_g2_body(alo, ahi, glo, ghi, dinv, b1, w2, olo, ohi):
    dv = dinv[...]
    h = jnp.concatenate([alo[...] + glo[...], ahi[...] + ghi[...]], axis=1)
    h = jnp.maximum(h * dv[:, None] + b1[...], 0.0)
    t = jnp.dot(h, w2[...], preferred_element_type=jnp.float32,
                precision=lax.Precision.HIGHEST)
    t = t * dv[:, None]
    olo[...] = t[:, :HALF]
    ohi[...] = t[:, HALF:]


def _tc_g2(alo, ahi, glo, ghi, dinv, b1r, W2):
    grid = (NP // BLK,)
    half_spec = pl.BlockSpec((BLK, HALF), lambda j: (j, 0))
    return pl.pallas_call(
        _tc_g2_body,
        grid=grid,
        in_specs=[
            half_spec, half_spec, half_spec, half_spec,
            pl.BlockSpec((BLK,), lambda j: (j,)),
            pl.BlockSpec((1, D), lambda j: (0, 0)),
            pl.BlockSpec((D, D), lambda j: (0, 0)),
        ],
        out_specs=[half_spec, half_spec],
        out_shape=[
            jax.ShapeDtypeStruct((NP, HALF), jnp.float32),
            jax.ShapeDtypeStruct((NP, HALF), jnp.float32),
        ],
        name="tc_g2",
    )(alo, ahi, glo, ghi, dinv, b1r, W2)


def _tc_q_body(alo, ahi, glo, ghi, dinv, b2, wl, q):
    dv = dinv[...]
    h = jnp.concatenate([alo[...] + glo[...], ahi[...] + ghi[...]], axis=1)
    h = jnp.maximum(h * dv[:, None] + b2[...], 0.0)
    w = wl[...]
    q[0, :] = jnp.sum(h * w[:, 0][None, :], axis=1)
    q[1, :] = jnp.sum(h * w[:, 1][None, :], axis=1)


def _tc_q(alo, ahi, glo, ghi, dinv, b2r, Wl):
    grid = (NP // BLK,)
    half_spec = pl.BlockSpec((BLK, HALF), lambda j: (j, 0))
    return pl.pallas_call(
        _tc_q_body,
        grid=grid,
        in_specs=[
            half_spec, half_spec, half_spec, half_spec,
            pl.BlockSpec((BLK,), lambda j: (j,)),
            pl.BlockSpec((1, D), lambda j: (0, 0)),
            pl.BlockSpec((D, NCL), lambda j: (0, 0)),
        ],
        out_specs=pl.BlockSpec((NCL, BLK), lambda j: (0, j)),
        out_shape=jax.ShapeDtypeStruct((NCL, NP), jnp.float32),
        name="tc_q",
    )(alo, ahi, glo, ghi, dinv, b2r, Wl)


@jax.jit
def _run(x, edge_index, batch, emb, W1, b1, W2, b2, Wl, bl):
    src = edge_index[0].astype(jnp.int32)
    dst = edge_index[1].astype(jnp.int32)
    xi = x.astype(jnp.int32)
    bi = batch.astype(jnp.int32)

    pad_e = jnp.full((EP - E,), DUM, jnp.int32)
    src_p = jnp.concatenate([src, pad_e])
    dst_p = jnp.concatenate([dst, pad_e])
    src3 = src_p.reshape(NTILE, EC_T, CH)
    dst3 = dst_p.reshape(NTILE, EC_T, CH)
    dstA = dst_p.reshape(NTILE * NCORE, EC_T // 2, CH)
    x3 = jnp.concatenate([xi, jnp.zeros((NP - N,), jnp.int32)]).reshape(
        NTILE * NCORE, NC_W, CH)
    batch_p = jnp.concatenate([bi, jnp.full((NP - N,), G, jnp.int32)])
    b3 = batch_p.reshape(NTILE, NC_T, CH)
    bA = batch_p.reshape(NTILE * NCORE, NC_W, CH)

    z1 = jnp.zeros((NP,), jnp.float32)
    z2c = jnp.zeros((CH, HALF), jnp.float32)
    b1r = b1.reshape(1, D)
    b2r = b2.reshape(1, D)
    blb = jnp.broadcast_to(bl[:, None], (NCL, 16))

    h0, deg2, cnt2 = _sc_prep(emb, x3, dstA, bA, z1)
    g1lo, g1hi, dinv = _tc_g1(deg2, h0, W1)
    a1lo, a1hi = _sc_prop(g1lo, g1hi, src3, dst3, z2c)
    g2lo, g2hi = _tc_g2(a1lo, a1hi, g1lo, g1hi, dinv, b1r, W2)
    a2lo, a2hi = _sc_prop(g2lo, g2hi, src3, dst3, z2c)
    q2 = _tc_q(a2lo, a2hi, g2lo, g2hi, dinv, b2r, Wl)
    res = _sc_pool(q2, b3, cnt2, blb)
    return res[:, :G].T


def kernel(x, edge_index, batch, emb, W1, b1, W2, b2, Wl, bl):
    return _run(x, edge_index, batch, emb, W1, b1, W2, b2, Wl, bl)


# speedup vs baseline: 27.5661x; 1.0002x over previous
import functools

import jax
import jax.numpy as jnp
from jax import lax
from jax.experimental import pallas as pl
from jax.experimental.pallas import tpu as pltpu
from jax.experimental.pallas import tpu_sc as plsc

N = 50000
E = 800000
G = 512
VOCAB = 100000
D = 64
NCL = 2

NTILE = 16
NCORE = 2
CH = 128
EC_T = 392
EP = NTILE * EC_T * CH
NP = 53248
NC_T = NP // CH // NTILE
NC_W = NP // CH // (NTILE * NCORE)
NPT = NP // NTILE
GP = 528
DUM = N
BLK = 1024
HALF = D // 2


EMB_BATCHES = (4, 4, 4, 1)


def _sc_prep_body(emb_h, x3_h, dstA_h, bA_h, z1_h,
                  h0_h, deg2_h, cnt2_h,
                  xv, rows, dstv, bv, ones_v, zg_v, deg_sh, cnt_sh,
                  dsem, gsem, wsem, hsem, csem):
    c = lax.axis_index("c")
    s = lax.axis_index("s")
    wid = c * NTILE + s

    pltpu.async_copy(dstA_h.at[wid], dstv, dsem)
    pltpu.sync_copy(x3_h.at[wid], xv)
    pltpu.sync_copy(bA_h.at[wid], bv)

    for i in range(CH // 16):
        ones_v[pl.ds(i * 16, 16)] = jnp.full((16,), 1.0, jnp.float32)

    pltpu.sync_copy(z1_h.at[pl.ds(s * NPT, NPT)], deg_sh.at[pl.ds(s * NPT, NPT)])

    @pl.when(s == 0)
    def _():
        for i in range(GP // 16):
            zg_v[pl.ds(i * 16, 16)] = jnp.zeros((16,), jnp.float32)
        pltpu.sync_copy(zg_v, cnt_sh)

    plsc.subcore_barrier()

    pltpu.make_async_copy(dstA_h.at[wid], dstv, dsem).wait()

    def deg_fire(k, _):
        pltpu.async_copy(ones_v, deg_sh.at[dstv.at[k]], hsem, add=True)
        return 0

    lax.fori_loop(0, EC_T // 2, deg_fire, 0)

    def cnt_fire(k, _):
        pltpu.async_copy(ones_v, cnt_sh.at[bv.at[k]], csem, add=True)
        return 0

    lax.fori_loop(0, NC_W, cnt_fire, 0)

    base = wid * NC_W * CH
    starts = [sum(EMB_BATCHES[:i]) for i in range(len(EMB_BATCHES))]

    def fire_gathers(bi):
        for j in range(EMB_BATCHES[bi]):
            k = starts[bi] + j
            pltpu.async_copy(emb_h.at[xv.at[k]], rows.at[k % 8], gsem)

    def drain_writebacks(bi):
        for j in range(EMB_BATCHES[bi]):
            k = starts[bi] + j
            pltpu.make_async_copy(rows.at[k % 8],
                                  h0_h.at[pl.ds(base + k * CH, CH)],
                                  wsem).wait()

    fire_gathers(0)
    for bi in range(len(EMB_BATCHES)):
        if bi >= 1:
            drain_writebacks(bi - 1)
        if bi + 1 < len(EMB_BATCHES):
            fire_gathers(bi + 1)
        for j in range(EMB_BATCHES[bi]):
            k = starts[bi] + j
            pltpu.make_async_copy(emb_h.at[xv.at[k]], rows.at[k % 8],
                                  gsem).wait()
        for j in range(EMB_BATCHES[bi]):
            k = starts[bi] + j
            pltpu.async_copy(rows.at[k % 8],
                             h0_h.at[pl.ds(base + k * CH, CH)], wsem)
    drain_writebacks(len(EMB_BATCHES) - 1)

    def deg_drain(k, _):
        pltpu.make_async_copy(ones_v, deg_sh.at[dstv.at[k]], hsem).wait()
        return 0

    lax.fori_loop(0, EC_T // 2, deg_drain, 0)

    def cnt_drain(k, _):
        pltpu.make_async_copy(ones_v, cnt_sh.at[bv.at[k]], csem).wait()
        return 0

    lax.fori_loop(0, NC_W, cnt_drain, 0)

    plsc.subcore_barrier()

    pltpu.sync_copy(deg_sh.at[pl.ds(s * NPT, NPT)],
                    deg2_h.at[c, pl.ds(s * NPT, NPT)])

    @pl.when(s == 0)
    def _():
        pltpu.sync_copy(cnt_sh, cnt2_h.at[c])


def _sc_prep(emb, x3, dstA, bA, z1):
    mesh = plsc.VectorSubcoreMesh(core_axis_name="c", subcore_axis_name="s")
    return pl.kernel(
        _sc_prep_body,
        out_type=[
            jax.ShapeDtypeStruct((NP, D), jnp.float32),
            jax.ShapeDtypeStruct((NCORE, NP), jnp.float32),
            jax.ShapeDtypeStruct((NCORE, GP), jnp.float32),
        ],
        mesh=mesh,
        scratch_types=[
            pltpu.VMEM((NC_W, CH), jnp.int32),
            pltpu.VMEM((8, CH, D), jnp.float32),
            pltpu.VMEM((EC_T // 2, CH), jnp.int32),
            pltpu.VMEM((NC_W, CH), jnp.int32),
            pltpu.VMEM((CH,), jnp.float32),
            pltpu.VMEM((GP,), jnp.float32),
            pltpu.VMEM_SHARED((NP,), jnp.float32),
            pltpu.VMEM_SHARED((GP,), jnp.float32),
            pltpu.SemaphoreType.DMA,
            pltpu.SemaphoreType.DMA,
            pltpu.SemaphoreType.DMA,
            pltpu.SemaphoreType.DMA,
            pltpu.SemaphoreType.DMA,
        ],
        name="sc_prep",
        compiler_params=pltpu.CompilerParams(use_tc_tiling_on_sc=False),
    )(emb, x3, dstA, bA, z1)


IB = 8
NG = EC_T // IB
NRB = 4


def _sc_prop_body(glo_h, ghi_h, src3_h, dst3_h, z2c_h,
                  alo_h, ahi_h,
                  sbuf, dbuf, r0, r1, r2, r3, acc_sh,
                  gs0, gs1, gs2, gs3, isem):
    c = lax.axis_index("c")
    s = lax.axis_index("s")
    rbufs = (r0, r1, r2, r3)
    gsems = (gs0, gs1, gs2, gs3)

    pltpu.sync_copy(z2c_h, r0)

    def zero_body(i, _):
        pltpu.sync_copy(r0, acc_sh.at[pl.ds(s * NPT + i * CH, CH)])
        return 0

    lax.fori_loop(0, NPT // CH, zero_body, 0)
    plsc.subcore_barrier()

    def run(g_h, a_h):
        for g0 in range(2):
            pltpu.async_copy(src3_h.at[s, pl.ds(g0 * IB, IB)], sbuf.at[g0], isem)
            pltpu.async_copy(dst3_h.at[s, pl.ds(g0 * IB, IB)], dbuf.at[g0], isem)
        for g0 in range(2):
            pltpu.make_async_copy(src3_h.at[s, pl.ds(g0 * IB, IB)],
                                  sbuf.at[g0], isem).wait()
            pltpu.make_async_copy(dst3_h.at[s, pl.ds(g0 * IB, IB)],
                                  dbuf.at[g0], isem).wait()

        for b in range(NRB):
            pltpu.async_copy(g_h.at[sbuf.at[0, b]], rbufs[b], gsems[b])

        def group(g, _):
            buf = lax.rem(g, 3)

            @pl.when((g >= 1) & (g + 1 < NG))
            def _():
                nb = lax.rem(g + 1, 3)
                pltpu.make_async_copy(src3_h.at[s, pl.ds((g + 1) * IB, IB)],
                                      sbuf.at[nb], isem).wait()
                pltpu.make_async_copy(dst3_h.at[s, pl.ds((g + 1) * IB, IB)],
                                      dbuf.at[nb], isem).wait()

            @pl.when(g + 2 < NG)
            def _():
                nb2 = lax.rem(g + 2, 3)
                pltpu.async_copy(src3_h.at[s, pl.ds((g + 2) * IB, IB)],
                                 sbuf.at[nb2], isem)
                pltpu.async_copy(dst3_h.at[s, pl.ds((g + 2) * IB, IB)],
                                 dbuf.at[nb2], isem)

            for b in range(IB):
                rb = rbufs[b % NRB]
                sem = gsems[b % NRB]
                pltpu.make_async_copy(g_h.at[sbuf.at[buf, b]], rb, sem).wait()
                pltpu.sync_copy(rb, acc_sh.at[dbuf.at[buf, b]], add=True)
                kk = b + NRB
                if kk < IB:
                    pltpu.async_copy(g_h.at[sbuf.at[buf, kk]], rb, sem)
                else:
                    @pl.when(g + 1 < NG)
                    def _(rb=rb, sem=sem, kk=kk):
                        nb = lax.rem(g + 1, 3)
                        pltpu.async_copy(g_h.at[sbuf.at[nb, kk - IB]], rb, sem)
            return 0

        lax.fori_loop(0, NG, group, 0)
        plsc.subcore_barrier()
        pltpu.sync_copy(acc_sh.at[pl.ds(s * NPT, NPT)],
                        a_h.at[pl.ds(s * NPT, NPT)])

    @pl.when(c == 0)
    def _():
        run(glo_h, alo_h)

    @pl.when(c == 1)
    def _():
        run(ghi_h, ahi_h)


def _sc_prop(g_lo, g_hi, src3, dst3, z2c):
    mesh = plsc.VectorSubcoreMesh(core_axis_name="c", subcore_axis_name="s")
    return pl.kernel(
        _sc_prop_body,
        out_type=[
            jax.ShapeDtypeStruct((NP, HALF), jnp.float32),
            jax.ShapeDtypeStruct((NP, HALF), jnp.float32),
        ],
        mesh=mesh,
        scratch_types=[
            pltpu.VMEM((3, IB, CH), jnp.int32),
            pltpu.VMEM((3, IB, CH), jnp.int32),
            pltpu.VMEM((CH, HALF), jnp.float32),
            pltpu.VMEM((CH, HALF), jnp.float32),
            pltpu.VMEM((CH, HALF), jnp.float32),
            pltpu.VMEM((CH, HALF), jnp.float32),
            pltpu.VMEM_SHARED((NP, HALF), jnp.float32),
            pltpu.SemaphoreType.DMA,
            pltpu.SemaphoreType.DMA,
            pltpu.SemaphoreType.DMA,
            pltpu.SemaphoreType.DMA,
            pltpu.SemaphoreType.DMA,
        ],
        name="sc_prop",
        compiler_params=pltpu.CompilerParams(use_tc_tiling_on_sc=False),
    )(g_lo, g_hi, src3, dst3, z2c)


def _sc_pool_body(q2_h, b3_h, cnt2_h, blb_h, res_h,
                  bv, qv, sums_v, ca, cb, blv, res_v, sums_sh):
    c = lax.axis_index("c")
    s = lax.axis_index("s")

    @pl.when(s == 0)
    def _():
        for i in range(GP // 16):
            res_v[pl.ds(i * 16, 16)] = jnp.zeros((16,), jnp.float32)
        pltpu.sync_copy(res_v, sums_sh)

    pltpu.sync_copy(q2_h.at[c, pl.ds(s * NPT, NPT)], qv)
    pltpu.sync_copy(b3_h.at[s], bv)
    plsc.subcore_barrier()

    def body(k, _):
        pltpu.sync_copy(qv.at[pl.ds(k * CH, CH)], sums_sh.at[bv.at[k]], add=True)
        return 0

    lax.fori_loop(0, NC_T, body, 0)
    plsc.subcore_barrier()

    @pl.when(s == 0)
    def _():
        pltpu.sync_copy(sums_sh, sums_v)
        pltpu.sync_copy(cnt2_h.at[0], ca)
        pltpu.sync_copy(cnt2_h.at[1], cb)
        pltpu.sync_copy(blb_h.at[c], blv)
        bvec = blv[...]
        for i in range(GP // 16):
            d = pl.ds(i * 16, 16)
            cv = ca[d] + cb[d]
            res_v[d] = sums_v[d] / jnp.maximum(cv, 1.0) + bvec
        pltpu.sync_copy(res_v, res_h.at[c])


def _sc_pool(q2, b3, cnt2, blb):
    mesh = plsc.VectorSubcoreMesh(core_axis_name="c", subcore_axis_name="s")
    return pl.kernel(
        _sc_pool_body,
        out_type=jax.ShapeDtypeStruct((NCORE, GP), jnp.float32),
        mesh=mesh,
        scratch_types=[
            pltpu.VMEM((NC_T, CH), jnp.int32),
            pltpu.VMEM((NPT,), jnp.float32),
            pltpu.VMEM((GP,), jnp.float32),
            pltpu.VMEM((GP,), jnp.float32),
            pltpu.VMEM((GP,), jnp.float32),
            pltpu.VMEM((16,), jnp.float32),
            pltpu.VMEM((GP,), jnp.float32),
            pltpu.VMEM_SHARED((GP,), jnp.float32),
        ],
        name="sc_pool",
        compiler_params=pltpu.CompilerParams(use_tc_tiling_on_sc=False),
    )(q2, b3, cnt2, blb)


def _tc_g1_body(deg2, h0, w1, glo, ghi, dinv):
    deg = deg2[0, :] + deg2[1, :] + 1.0
    dv = lax.rsqrt(deg)
    t = jnp.dot(h0[...], w1[...], preferred_element_type=jnp.float32,
                precision=lax.Precision.HIGHEST)
    t = t * dv[:, None]
    glo[...] = t[:, :HALF]
    ghi[...] = t[:, HALF:]
    dinv[...] = dv


def _tc_g1(deg2, h0, W1):
    grid = (NP // BLK,)
    hf_spec = pl.BlockSpec((BLK, HALF), lambda j: (j, 0))
    return pl.pallas_call(
        _tc_g1_body,
        grid=grid,
        in_specs=[
            pl.BlockSpec((NCORE, BLK), lambda j: (0, j)),
            pl.BlockSpec((BLK, D), lambda j: (j, 0)),
            pl.BlockSpec((D, D), lambda j: (0, 0)),
        ],
        out_specs=[
            hf_spec,
            hf_spec,
            pl.BlockSpec((BLK,), lambda j: (j,)),
        ],
        out_shape=[
            jax.ShapeDtypeStruct((NP, HALF), jnp.float32),
            jax.ShapeDtypeStruct((NP, HALF), jnp.float32),
            jax.ShapeDtypeStruct((NP,), jnp.float32),
        ],
        name="tc_g1",
    )(deg2, h0, W1)


def _tc_g2_body(alo, ahi, glo, ghi, dinv, b1, w2, olo, ohi):
    dv = dinv[...]
    lo = alo[...] + glo[...]
    hi = ahi[...] + ghi[...]
    h = jnp.concatenate([lo, hi], axis=1)
    h = jnp.maximum(h * dv[:, None] + b1[...], 0.0)
    t = jnp.dot(h, w2[...], preferred_element_type=jnp.float32,
                precision=lax.Precision.HIGHEST)
    t = t * dv[:, None]
    olo[...] = t[:, :HALF]
    ohi[...] = t[:, HALF:]


def _tc_g2(alo, ahi, glo, ghi, dinv, b1r, W2):
    grid = (NP // BLK,)
    hf_spec = pl.BlockSpec((BLK, HALF), lambda j: (j, 0))
    return pl.pallas_call(
        _tc_g2_body,
        grid=grid,
        in_specs=[
            hf_spec, hf_spec, hf_spec, hf_spec,
            pl.BlockSpec((BLK,), lambda j: (j,)),
            pl.BlockSpec((1, D), lambda j: (0, 0)),
            pl.BlockSpec((D, D), lambda j: (0, 0)),
        ],
        out_specs=[hf_spec, hf_spec],
        out_shape=[
            jax.ShapeDtypeStruct((NP, HALF), jnp.float32),
            jax.ShapeDtypeStruct((NP, HALF), jnp.float32),
        ],
        name="tc_g2",
    )(alo, ahi, glo, ghi, dinv, b1r, W2)


def _tc_q_body(alo, ahi, glo, ghi, dinv, b2, wl, q):
    dv = dinv[...]
    lo = alo[...] + glo[...]
    hi = ahi[...] + ghi[...]
    h = jnp.concatenate([lo, hi], axis=1)
    h = jnp.maximum(h * dv[:, None] + b2[...], 0.0)
    w = wl[...]
    q[0, :] = jnp.sum(h * w[:, 0][None, :], axis=1)
    q[1, :] = jnp.sum(h * w[:, 1][None, :], axis=1)


def _tc_q(alo, ahi, glo, ghi, dinv, b2r, Wl):
    grid = (NP // BLK,)
    hf_spec = pl.BlockSpec((BLK, HALF), lambda j: (j, 0))
    return pl.pallas_call(
        _tc_q_body,
        grid=grid,
        in_specs=[
            hf_spec, hf_spec, hf_spec, hf_spec,
            pl.BlockSpec((BLK,), lambda j: (j,)),
            pl.BlockSpec((1, D), lambda j: (0, 0)),
            pl.BlockSpec((D, NCL), lambda j: (0, 0)),
        ],
        out_specs=pl.BlockSpec((NCL, BLK), lambda j: (0, j)),
        out_shape=jax.ShapeDtypeStruct((NCL, NP), jnp.float32),
        name="tc_q",
    )(alo, ahi, glo, ghi, dinv, b2r, Wl)


@jax.jit
def _run(x, edge_index, batch, emb, W1, b1, W2, b2, Wl, bl):
    src = edge_index[0].astype(jnp.int32)
    dst = edge_index[1].astype(jnp.int32)
    xi = x.astype(jnp.int32)
    bi = batch.astype(jnp.int32)

    pad_e = jnp.full((EP - E,), DUM, jnp.int32)
    src_p = jnp.concatenate([src, pad_e])
    dst_p = jnp.concatenate([dst, pad_e])
    src3 = src_p.reshape(NTILE, EC_T, CH)
    dst3 = dst_p.reshape(NTILE, EC_T, CH)
    dstA = dst_p.reshape(NTILE * NCORE, EC_T // 2, CH)
    x3 = jnp.concatenate([xi, jnp.zeros((NP - N,), jnp.int32)]).reshape(
        NTILE * NCORE, NC_W, CH)
    batch_p = jnp.concatenate([bi, jnp.full((NP - N,), G, jnp.int32)])
    b3 = batch_p.reshape(NTILE, NC_T, CH)
    bA = batch_p.reshape(NTILE * NCORE, NC_W, CH)

    z1 = jnp.zeros((NP,), jnp.float32)
    z2c = jnp.zeros((CH, HALF), jnp.float32)
    b1r = b1.reshape(1, D)
    b2r = b2.reshape(1, D)
    blb = jnp.broadcast_to(bl[:, None], (NCL, 16))

    h0, deg2, cnt2 = _sc_prep(emb, x3, dstA, bA, z1)
    g1lo, g1hi, dinv = _tc_g1(deg2, h0, W1)
    a1lo, a1hi = _sc_prop(g1lo, g1hi, src3, dst3, z2c)
    g2lo, g2hi = _tc_g2(a1lo, a1hi, g1lo, g1hi, dinv, b1r, W2)
    a2lo, a2hi = _sc_prop(g2lo, g2hi, src3, dst3, z2c)
    q2 = _tc_q(a2lo, a2hi, g2lo, g2hi, dinv, b2r, Wl)
    res = _sc_pool(q2, b3, cnt2, blb)
    return res[:, :G].T


def kernel(x, edge_index, batch, emb, W1, b1, W2, b2, Wl, bl):
    return _run(x, edge_index, batch, emb, W1, b1, W2, b2, Wl, bl)


# speedup vs baseline: 27.7786x; 1.0077x over previous
import functools

import jax
import jax.numpy as jnp
from jax import lax
from jax.experimental import pallas as pl
from jax.experimental.pallas import tpu as pltpu
from jax.experimental.pallas import tpu_sc as plsc

N = 50000
E = 800000
G = 512
VOCAB = 100000
D = 64
NCL = 2

NTILE = 16
NCORE = 2
CH = 128
EC_T = 392
EP = NTILE * EC_T * CH
NP = 53248
NC_T = NP // CH // NTILE
NC_W = NP // CH // (NTILE * NCORE)
NPT = NP // NTILE
GP = 528
DUM = N
BLK = 1024
HALF = D // 2


EMB_BATCHES = (4, 4, 4, 1)


def _sc_prep_body(emb_h, x3_h, dstA_h, bA_h, z1_h,
                  h0_h, deg2_h, cnt2_h,
                  xv, rows, dstv, bv, ones_v, zg_v, deg_sh, cnt_sh,
                  dsem, gsem, wsem, hsem, csem):
    c = lax.axis_index("c")
    s = lax.axis_index("s")
    wid = c * NTILE + s

    pltpu.async_copy(dstA_h.at[wid], dstv, dsem)
    pltpu.sync_copy(x3_h.at[wid], xv)
    pltpu.sync_copy(bA_h.at[wid], bv)

    for i in range(CH // 16):
        ones_v[pl.ds(i * 16, 16)] = jnp.full((16,), 1.0, jnp.float32)

    pltpu.sync_copy(z1_h.at[pl.ds(s * NPT, NPT)], deg_sh.at[pl.ds(s * NPT, NPT)])

    @pl.when(s == 0)
    def _():
        for i in range(GP // 16):
            zg_v[pl.ds(i * 16, 16)] = jnp.zeros((16,), jnp.float32)
        pltpu.sync_copy(zg_v, cnt_sh)

    plsc.subcore_barrier()

    pltpu.make_async_copy(dstA_h.at[wid], dstv, dsem).wait()

    def deg_fire(k, _):
        pltpu.async_copy(ones_v, deg_sh.at[dstv.at[k]], hsem, add=True)
        return 0

    lax.fori_loop(0, EC_T // 2, deg_fire, 0)

    def cnt_fire(k, _):
        pltpu.async_copy(ones_v, cnt_sh.at[bv.at[k]], csem, add=True)
        return 0

    lax.fori_loop(0, NC_W, cnt_fire, 0)

    base = wid * NC_W * CH
    starts = [sum(EMB_BATCHES[:i]) for i in range(len(EMB_BATCHES))]

    def fire_gathers(bi):
        for j in range(EMB_BATCHES[bi]):
            k = starts[bi] + j
            pltpu.async_copy(emb_h.at[xv.at[k]], rows.at[k % 8], gsem)

    def drain_writebacks(bi):
        for j in range(EMB_BATCHES[bi]):
            k = starts[bi] + j
            pltpu.make_async_copy(rows.at[k % 8],
                                  h0_h.at[pl.ds(base + k * CH, CH)],
                                  wsem).wait()

    fire_gathers(0)
    for bi in range(len(EMB_BATCHES)):
        if bi >= 1:
            drain_writebacks(bi - 1)
        if bi + 1 < len(EMB_BATCHES):
            fire_gathers(bi + 1)
        for j in range(EMB_BATCHES[bi]):
            k = starts[bi] + j
            pltpu.make_async_copy(emb_h.at[xv.at[k]], rows.at[k % 8],
                                  gsem).wait()
        for j in range(EMB_BATCHES[bi]):
            k = starts[bi] + j
            pltpu.async_copy(rows.at[k % 8],
                             h0_h.at[pl.ds(base + k * CH, CH)], wsem)
    drain_writebacks(len(EMB_BATCHES) - 1)

    def deg_drain(k, _):
        pltpu.make_async_copy(ones_v, deg_sh.at[dstv.at[k]], hsem).wait()
        return 0

    lax.fori_loop(0, EC_T // 2, deg_drain, 0)

    def cnt_drain(k, _):
        pltpu.make_async_copy(ones_v, cnt_sh.at[bv.at[k]], csem).wait()
        return 0

    lax.fori_loop(0, NC_W, cnt_drain, 0)

    plsc.subcore_barrier()

    pltpu.sync_copy(deg_sh.at[pl.ds(s * NPT, NPT)],
                    deg2_h.at[c, pl.ds(s * NPT, NPT)])

    @pl.when(s == 0)
    def _():
        pltpu.sync_copy(cnt_sh, cnt2_h.at[c])


def _sc_prep(emb, x3, dstA, bA, z1):
    mesh = plsc.VectorSubcoreMesh(core_axis_name="c", subcore_axis_name="s")
    return pl.kernel(
        _sc_prep_body,
        out_type=[
            jax.ShapeDtypeStruct((NP, D), jnp.float32),
            jax.ShapeDtypeStruct((NCORE, NP), jnp.float32),
            jax.ShapeDtypeStruct((NCORE, GP), jnp.float32),
        ],
        mesh=mesh,
        scratch_types=[
            pltpu.VMEM((NC_W, CH), jnp.int32),
            pltpu.VMEM((8, CH, D), jnp.float32),
            pltpu.VMEM((EC_T // 2, CH), jnp.int32),
            pltpu.VMEM((NC_W, CH), jnp.int32),
            pltpu.VMEM((CH,), jnp.float32),
            pltpu.VMEM((GP,), jnp.float32),
            pltpu.VMEM_SHARED((NP,), jnp.float32),
            pltpu.VMEM_SHARED((GP,), jnp.float32),
            pltpu.SemaphoreType.DMA,
            pltpu.SemaphoreType.DMA,
            pltpu.SemaphoreType.DMA,
            pltpu.SemaphoreType.DMA,
            pltpu.SemaphoreType.DMA,
        ],
        name="sc_prep",
        compiler_params=pltpu.CompilerParams(use_tc_tiling_on_sc=False),
    )(emb, x3, dstA, bA, z1)


IB = 8
NG = EC_T // IB
NRB = 4


def _sc_prop_body(glo_h, ghi_h, src3_h, dst3_h, z2c_h,
                  alo_h, ahi_h,
                  sbuf, dbuf, r0, r1, r2, r3, acc_sh,
                  gs0, gs1, gs2, gs3, isem):
    c = lax.axis_index("c")
    s = lax.axis_index("s")
    rbufs = (r0, r1, r2, r3)
    gsems = (gs0, gs1, gs2, gs3)

    pltpu.sync_copy(z2c_h, r0)

    def zero_body(i, _):
        pltpu.sync_copy(r0, acc_sh.at[pl.ds(s * NPT + i * CH, CH)])
        return 0

    lax.fori_loop(0, NPT // CH, zero_body, 0)
    plsc.subcore_barrier()

    def run(g_h, a_h):
        for g0 in range(2):
            pltpu.async_copy(src3_h.at[s, pl.ds(g0 * IB, IB)], sbuf.at[g0], isem)
            pltpu.async_copy(dst3_h.at[s, pl.ds(g0 * IB, IB)], dbuf.at[g0], isem)
        for g0 in range(2):
            pltpu.make_async_copy(src3_h.at[s, pl.ds(g0 * IB, IB)],
                                  sbuf.at[g0], isem).wait()
            pltpu.make_async_copy(dst3_h.at[s, pl.ds(g0 * IB, IB)],
                                  dbuf.at[g0], isem).wait()

        for b in range(NRB):
            pltpu.async_copy(g_h.at[sbuf.at[0, b]], rbufs[b], gsems[b])

        def group(g, _):
            buf = lax.rem(g, 3)

            @pl.when((g >= 1) & (g + 1 < NG))
            def _():
                nb = lax.rem(g + 1, 3)
                pltpu.make_async_copy(src3_h.at[s, pl.ds((g + 1) * IB, IB)],
                                      sbuf.at[nb], isem).wait()
                pltpu.make_async_copy(dst3_h.at[s, pl.ds((g + 1) * IB, IB)],
                                      dbuf.at[nb], isem).wait()

            @pl.when(g + 2 < NG)
            def _():
                nb2 = lax.rem(g + 2, 3)
                pltpu.async_copy(src3_h.at[s, pl.ds((g + 2) * IB, IB)],
                                 sbuf.at[nb2], isem)
                pltpu.async_copy(dst3_h.at[s, pl.ds((g + 2) * IB, IB)],
                                 dbuf.at[nb2], isem)

            for b in range(IB):
                rb = rbufs[b % NRB]
                sem = gsems[b % NRB]
                pltpu.make_async_copy(g_h.at[sbuf.at[buf, b]], rb, sem).wait()
                pltpu.sync_copy(rb, acc_sh.at[dbuf.at[buf, b]], add=True)
                kk = b + NRB
                if kk < IB:
                    pltpu.async_copy(g_h.at[sbuf.at[buf, kk]], rb, sem)
                else:
                    @pl.when(g + 1 < NG)
                    def _(rb=rb, sem=sem, kk=kk):
                        nb = lax.rem(g + 1, 3)
                        pltpu.async_copy(g_h.at[sbuf.at[nb, kk - IB]], rb, sem)
            return 0

        lax.fori_loop(0, NG, group, 0)
        plsc.subcore_barrier()
        pltpu.sync_copy(acc_sh.at[pl.ds(s * NPT, NPT)],
                        a_h.at[pl.ds(s * NPT, NPT)])

    @pl.when(c == 0)
    def _():
        run(glo_h, alo_h)

    @pl.when(c == 1)
    def _():
        run(ghi_h, ahi_h)


def _sc_prop(g_lo, g_hi, src3, dst3, z2c):
    mesh = plsc.VectorSubcoreMesh(core_axis_name="c", subcore_axis_name="s")
    return pl.kernel(
        _sc_prop_body,
        out_type=[
            jax.ShapeDtypeStruct((NP, HALF), jnp.float32),
            jax.ShapeDtypeStruct((NP, HALF), jnp.float32),
        ],
        mesh=mesh,
        scratch_types=[
            pltpu.VMEM((3, IB, CH), jnp.int32),
            pltpu.VMEM((3, IB, CH), jnp.int32),
            pltpu.VMEM((CH, HALF), jnp.float32),
            pltpu.VMEM((CH, HALF), jnp.float32),
            pltpu.VMEM((CH, HALF), jnp.float32),
            pltpu.VMEM((CH, HALF), jnp.float32),
            pltpu.VMEM_SHARED((NP, HALF), jnp.float32),
            pltpu.SemaphoreType.DMA,
            pltpu.SemaphoreType.DMA,
            pltpu.SemaphoreType.DMA,
            pltpu.SemaphoreType.DMA,
            pltpu.SemaphoreType.DMA,
        ],
        name="sc_prop",
        compiler_params=pltpu.CompilerParams(use_tc_tiling_on_sc=False),
    )(g_lo, g_hi, src3, dst3, z2c)


def _sc_pool_body(q2_h, b3_h, cnt2_h, blb_h, res_h,
                  bv, qv, sums_v, ca, cb, blv, res_v, sums_sh):
    c = lax.axis_index("c")
    s = lax.axis_index("s")

    @pl.when(s == 0)
    def _():
        for i in range(GP // 16):
            res_v[pl.ds(i * 16, 16)] = jnp.zeros((16,), jnp.float32)
        pltpu.sync_copy(res_v, sums_sh)

    pltpu.sync_copy(q2_h.at[c, pl.ds(s * NPT, NPT)], qv)
    pltpu.sync_copy(b3_h.at[s], bv)
    plsc.subcore_barrier()

    def body(k, _):
        pltpu.sync_copy(qv.at[pl.ds(k * CH, CH)], sums_sh.at[bv.at[k]], add=True)
        return 0

    lax.fori_loop(0, NC_T, body, 0)
    plsc.subcore_barrier()

    @pl.when(s == 0)
    def _():
        pltpu.sync_copy(sums_sh, sums_v)
        pltpu.sync_copy(cnt2_h.at[0], ca)
        pltpu.sync_copy(cnt2_h.at[1], cb)
        pltpu.sync_copy(blb_h.at[c], blv)
        bvec = blv[...]
        for i in range(GP // 16):
            d = pl.ds(i * 16, 16)
            cv = ca[d] + cb[d]
            res_v[d] = sums_v[d] / jnp.maximum(cv, 1.0) + bvec
        pltpu.sync_copy(res_v, res_h.at[c])


def _sc_pool(q2, b3, cnt2, blb):
    mesh = plsc.VectorSubcoreMesh(core_axis_name="c", subcore_axis_name="s")
    return pl.kernel(
        _sc_pool_body,
        out_type=jax.ShapeDtypeStruct((NCORE, GP), jnp.float32),
        mesh=mesh,
        scratch_types=[
            pltpu.VMEM((NC_T, CH), jnp.int32),
            pltpu.VMEM((NPT,), jnp.float32),
            pltpu.VMEM((GP,), jnp.float32),
            pltpu.VMEM((GP,), jnp.float32),
            pltpu.VMEM((GP,), jnp.float32),
            pltpu.VMEM((16,), jnp.float32),
            pltpu.VMEM((GP,), jnp.float32),
            pltpu.VMEM_SHARED((GP,), jnp.float32),
        ],
        name="sc_pool",
        compiler_params=pltpu.CompilerParams(use_tc_tiling_on_sc=False),
    )(q2, b3, cnt2, blb)


def _tc_g1_body(deg2, h0, w1, glo, ghi, dinv):
    deg = deg2[0, :] + deg2[1, :] + 1.0
    dv = lax.rsqrt(deg)
    t = jnp.dot(h0[...], w1[...], preferred_element_type=jnp.float32)
    t = t * dv[:, None]
    glo[...] = t[:, :HALF]
    ghi[...] = t[:, HALF:]
    dinv[...] = dv


def _tc_g1(deg2, h0, W1):
    grid = (NP // BLK,)
    hf_spec = pl.BlockSpec((BLK, HALF), lambda j: (j, 0))
    return pl.pallas_call(
        _tc_g1_body,
        grid=grid,
        in_specs=[
            pl.BlockSpec((NCORE, BLK), lambda j: (0, j)),
            pl.BlockSpec((BLK, D), lambda j: (j, 0)),
            pl.BlockSpec((D, D), lambda j: (0, 0)),
        ],
        out_specs=[
            hf_spec,
            hf_spec,
            pl.BlockSpec((BLK,), lambda j: (j,)),
        ],
        out_shape=[
            jax.ShapeDtypeStruct((NP, HALF), jnp.float32),
            jax.ShapeDtypeStruct((NP, HALF), jnp.float32),
            jax.ShapeDtypeStruct((NP,), jnp.float32),
        ],
        name="tc_g1",
    )(deg2, h0, W1)


def _tc_g2_body(alo, ahi, glo, ghi, dinv, b1, w2, olo, ohi):
    dv = dinv[...]
    lo = alo[...] + glo[...]
    hi = ahi[...] + ghi[...]
    h = jnp.concatenate([lo, hi], axis=1)
    h = jnp.maximum(h * dv[:, None] + b1[...], 0.0)
    t = jnp.dot(h, w2[...], preferred_element_type=jnp.float32)
    t = t * dv[:, None]
    olo[...] = t[:, :HALF]
    ohi[...] = t[:, HALF:]


def _tc_g2(alo, ahi, glo, ghi, dinv, b1r, W2):
    grid = (NP // BLK,)
    hf_spec = pl.BlockSpec((BLK, HALF), lambda j: (j, 0))
    return pl.pallas_call(
        _tc_g2_body,
        grid=grid,
        in_specs=[
            hf_spec, hf_spec, hf_spec, hf_spec,
            pl.BlockSpec((BLK,), lambda j: (j,)),
            pl.BlockSpec((1, D), lambda j: (0, 0)),
            pl.BlockSpec((D, D), lambda j: (0, 0)),
        ],
        out_specs=[hf_spec, hf_spec],
        out_shape=[
            jax.ShapeDtypeStruct((NP, HALF), jnp.float32),
            jax.ShapeDtypeStruct((NP, HALF), jnp.float32),
        ],
        name="tc_g2",
    )(alo, ahi, glo, ghi, dinv, b1r, W2)


def _tc_q_body(alo, ahi, glo, ghi, dinv, b2, wl, q):
    dv = dinv[...]
    lo = alo[...] + glo[...]
    hi = ahi[...] + ghi[...]
    h = jnp.concatenate([lo, hi], axis=1)
    h = jnp.maximum(h * dv[:, None] + b2[...], 0.0)
    w = wl[...]
    q[0, :] = jnp.sum(h * w[:, 0][None, :], axis=1)
    q[1, :] = jnp.sum(h * w[:, 1][None, :], axis=1)


def _tc_q(alo, ahi, glo, ghi, dinv, b2r, Wl):
    grid = (NP // BLK,)
    hf_spec = pl.BlockSpec((BLK, HALF), lambda j: (j, 0))
    return pl.pallas_call(
        _tc_q_body,
        grid=grid,
        in_specs=[
            hf_spec, hf_spec, hf_spec, hf_spec,
            pl.BlockSpec((BLK,), lambda j: (j,)),
            pl.BlockSpec((1, D), lambda j: (0, 0)),
            pl.BlockSpec((D, NCL), lambda j: (0, 0)),
        ],
        out_specs=pl.BlockSpec((NCL, BLK), lambda j: (0, j)),
        out_shape=jax.ShapeDtypeStruct((NCL, NP), jnp.float32),
        name="tc_q",
    )(alo, ahi, glo, ghi, dinv, b2r, Wl)


@jax.jit
def _run(x, edge_index, batch, emb, W1, b1, W2, b2, Wl, bl):
    src = edge_index[0].astype(jnp.int32)
    dst = edge_index[1].astype(jnp.int32)
    xi = x.astype(jnp.int32)
    bi = batch.astype(jnp.int32)

    pad_e = jnp.full((EP - E,), DUM, jnp.int32)
    src_p = jnp.concatenate([src, pad_e])
    dst_p = jnp.concatenate([dst, pad_e])
    src3 = src_p.reshape(NTILE, EC_T, CH)
    dst3 = dst_p.reshape(NTILE, EC_T, CH)
    dstA = dst_p.reshape(NTILE * NCORE, EC_T // 2, CH)
    x3 = jnp.concatenate([xi, jnp.zeros((NP - N,), jnp.int32)]).reshape(
        NTILE * NCORE, NC_W, CH)
    batch_p = jnp.concatenate([bi, jnp.full((NP - N,), G, jnp.int32)])
    b3 = batch_p.reshape(NTILE, NC_T, CH)
    bA = batch_p.reshape(NTILE * NCORE, NC_W, CH)

    z1 = jnp.zeros((NP,), jnp.float32)
    z2c = jnp.zeros((CH, HALF), jnp.float32)
    b1r = b1.reshape(1, D)
    b2r = b2.reshape(1, D)
    blb = jnp.broadcast_to(bl[:, None], (NCL, 16))

    h0, deg2, cnt2 = _sc_prep(emb, x3, dstA, bA, z1)
    g1lo, g1hi, dinv = _tc_g1(deg2, h0, W1)
    a1lo, a1hi = _sc_prop(g1lo, g1hi, src3, dst3, z2c)
    g2lo, g2hi = _tc_g2(a1lo, a1hi, g1lo, g1hi, dinv, b1r, W2)
    a2lo, a2hi = _sc_prop(g2lo, g2hi, src3, dst3, z2c)
    q2 = _tc_q(a2lo, a2hi, g2lo, g2hi, dinv, b2r, Wl)
    res = _sc_pool(q2, b3, cnt2, blb)
    return res[:, :G].T


def kernel(x, edge_index, batch, emb, W1, b1, W2, b2, Wl, bl):
    return _run(x, edge_index, batch, emb, W1, b1, W2, b2, Wl, bl)
